# baseline (speedup 1.0000x reference)
import functools
import numpy as np
import jax
import jax.numpy as jnp
from jax.experimental import pallas as pl
from jax.experimental.pallas import tpu as pltpu

_N_M, _N_D, _N_A = 20000, 4000, 16000
_IN = 128
_HID = 128
_HEADS = 4
_SVD = 64
_CLASSES = 8
_LT = 2
_DH = _HID // _HEADS


def _head_body(cat_ref, w0_ref, b0_ref, w1_ref, b1_ref, w2_ref,
               xgt_ref, out_ref):
    x = cat_ref[...]
    xg = jnp.maximum(x @ w0_ref[...] + b0_ref[...], 0.0)
    xgt_ref[...] = xg
    h = jnp.maximum(xg @ w1_ref[...] + b1_ref[...], 0.0)
    o = h @ w2_ref[...]
    o = o - jnp.max(o, axis=-1, keepdims=True)
    e = jnp.exp(o)
    out_ref[...] = e / jnp.sum(e, axis=-1, keepdims=True)


def _head(cat, w0, b0, w1, b1, w2):
    blk = 2000
    n = cat.shape[0]
    grid = (n // blk,)
    full = lambda *s: pl.BlockSpec(s, lambda i: tuple(0 for _ in s))
    xgt, out = pl.pallas_call(
        _head_body,
        grid=grid,
        in_specs=[
            pl.BlockSpec((blk, cat.shape[1]), lambda i: (i, 0)),
            full(*w0.shape), full(*b0.shape), full(*w1.shape),
            full(*b1.shape), full(*w2.shape),
        ],
        out_specs=[
            pl.BlockSpec((blk, _HID), lambda i: (i, 0)),
            pl.BlockSpec((blk, _CLASSES), lambda i: (i, 0)),
        ],
        out_shape=[
            jax.ShapeDtypeStruct((n, _HID), jnp.float32),
            jax.ShapeDtypeStruct((n, _CLASSES), jnp.float32),
        ],
    )(cat, w0, b0, w1, b1, w2)
    return xgt, out


def _seg_softmax(e, seg, n):
    m = jax.ops.segment_max(e, seg, num_segments=n)
    m = jnp.where(jnp.isfinite(m), m, 0.0)
    ex = jnp.exp(e - m[seg])
    s = jax.ops.segment_sum(ex, seg, num_segments=n)
    return ex / (s[seg] + 1e-16)


def _gat(h_src, h_dst, ei, W, a_s, a_d, heads, oc, n_dst):
    src, dst = ei[0], ei[1]
    hs = (h_src @ W).reshape(-1, heads, oc)
    hd = (h_dst @ W).reshape(-1, heads, oc)
    es = jnp.sum(hs * a_s, axis=-1)
    ed = jnp.sum(hd * a_d, axis=-1)
    e = jax.nn.leaky_relu(es[src] + ed[dst], 0.2)
    alpha = _seg_softmax(e, dst, n_dst)
    out = jax.ops.segment_sum(alpha[:, :, None] * hs[src], dst, num_segments=n_dst)
    return out.reshape(n_dst, heads * oc)


def _ln(x, g, b):
    mu = jnp.mean(x, axis=-1, keepdims=True)
    v = jnp.var(x, axis=-1, keepdims=True)
    return (x - mu) / jnp.sqrt(v + 1e-5) * g + b


def _gt_layer(h, peQ, peK, deg, p, l, ei):
    src, dst = ei[0], ei[1]
    Q = (h @ p['gt%d_Wq' % l] + peQ @ p['gt%d_Wpq' % l]).reshape(-1, _HEADS, _DH)
    K = (h @ p['gt%d_Wk' % l] + peK @ p['gt%d_Wpk' % l]).reshape(-1, _HEADS, _DH)
    V = (h @ p['gt%d_Wv' % l]).reshape(-1, _HEADS, _DH)
    sc = jnp.sum(Q[dst] * K[src], axis=-1) / np.sqrt(_DH)
    al = _seg_softmax(sc, dst, _N_M)
    agg = jax.ops.segment_sum(al[:, :, None] * V[src], dst, num_segments=_N_M).reshape(_N_M, _HID)
    agg = agg / jnp.sqrt(deg + 1.0)[:, None]
    h1 = _ln(h + agg @ p['gt%d_Wo' % l], p['gt%d_ln1g' % l], p['gt%d_ln1b' % l])
    ff = jax.nn.relu(h1 @ p['gt%d_W1' % l] + p['gt%d_b1' % l]) @ p['gt%d_W2' % l] + p['gt%d_b2' % l]
    return _ln(h1 + ff, p['gt%d_ln2g' % l], p['gt%d_ln2b' % l])


def kernel(x_movie, x_director, x_actor, deg, params,
           edge_index_md, edge_index_dm, edge_index_ma, edge_index_am,
           edge_index_meta):
    p = params
    xm = jax.nn.relu(x_movie @ p['t1_W'] + p['t1_b'])
    xd = jax.nn.relu(x_director @ p['t2_W'] + p['t2_b'])
    xa = jax.nn.relu(x_actor @ p['t3_W'] + p['t3_b'])
    m1 = jax.nn.relu(
        _gat(xd, xm, edge_index_dm, p['g1_dm_W'], p['g1_dm_as'], p['g1_dm_ad'], _HEADS, _HID, _N_M)
        + _gat(xa, xm, edge_index_am, p['g1_am_W'], p['g1_am_as'], p['g1_am_ad'], _HEADS, _HID, _N_M))
    d1 = jax.nn.relu(_gat(xm, xd, edge_index_md, p['g1_md_W'], p['g1_md_as'], p['g1_md_ad'], _HEADS, _HID, _N_D))
    a1 = jax.nn.relu(_gat(xm, xa, edge_index_ma, p['g1_ma_W'], p['g1_ma_as'], p['g1_ma_ad'], _HEADS, _HID, _N_A))
    m2 = (_gat(d1, m1, edge_index_dm, p['g2_dm_W'], p['g2_dm_as'], p['g2_dm_ad'], 1, _HID, _N_M)
          + _gat(a1, m1, edge_index_am, p['g2_am_W'], p['g2_am_as'], p['g2_am_ad'], 1, _HID, _N_M))
    peQ = p['peQ']; peK = p['peK']
    dots = jnp.sum(peQ[edge_index_meta[0]] * peK[edge_index_meta[1]], axis=-1) / np.sqrt(_SVD)
    loss_pe = jnp.mean((dots - 1.0) ** 2)
    save = [m2]
    h = m2
    for l in range(_LT):
        h = _gt_layer(h, peQ, peK, deg, p, l, edge_index_meta)
        save.append(h)
    cat = jnp.transpose(jnp.stack(save, axis=0), (1, 0, 2)).reshape(_N_M, -1)
    x_gt, out_logits = _head(cat, p['cat_W'], p['cat_b'], p['m1_W'], p['m1_b'], p['m2_W'])
    return (p['alpha'] * loss_pe, out_logits, x_gt, peQ, peK)
